# Initial kernel scaffold; baseline (speedup 1.0000x reference)
import jax
import jax.numpy as jnp
from jax.experimental import pallas as pl


def kernel(x, edge_index, W1, b1, g1, be1, W2, b2, g2, be2):
    raise NotImplementedError("write your pallas kernel here")


# speedup vs baseline: 14.0142x; 14.0142x over previous
import jax
import jax.numpy as jnp
from jax import lax
from jax.experimental import pallas as pl
from jax.experimental.pallas import tpu as pltpu
from jax.experimental.pallas import tpu_sc as plsc

N = 10000
E = 320000
D = 128
EPS = 1e-5

NC = 2
NS = 16
NW = NC * NS
EPW = E // NW
CH = 80
NCHUNK = EPW // CH
NBUF = 5
NGRP = NCHUNK // NBUF
NPAD = 10240
RPT = NPAD // NS


def _mesh():
    return plsc.VectorSubcoreMesh(core_axis_name="c", subcore_axis_name="s",
                                  num_cores=NC, num_subcores=NS)


def _sc_aggregate_body(hp_hbm, src_hbm, dst_hbm, out_hbm,
                       idx_s, idx_d, rows, agg, *sems):
    gsem, ssem = sems
    c = lax.axis_index("c")
    s = lax.axis_index("s")
    wid = s * NC + c

    def zrow(i, carry):
        for q in range(D // 16):
            rows[i, pl.ds(q * 16, 16)] = jnp.zeros((16,), jnp.float32)
        return carry

    lax.fori_loop(0, CH, zrow, 0)
    for q in range(RPT // CH):
        pltpu.sync_copy(rows, agg.at[pl.ds(s * RPT + q * CH, CH)])
    pltpu.sync_copy(src_hbm.at[wid], idx_s)
    pltpu.sync_copy(dst_hbm.at[wid], idx_d)
    plsc.subcore_barrier()

    def chunk(j, carry):
        pltpu.async_copy(hp_hbm.at[idx_s.at[j]], rows, gsem).wait()
        pltpu.async_copy(rows, agg.at[idx_d.at[j]], ssem,
                         add=True).wait()
        return carry

    lax.fori_loop(0, NCHUNK, chunk, 0)
    plsc.subcore_barrier()
    pltpu.sync_copy(agg.at[pl.ds(s * RPT, RPT)],
                    out_hbm.at[c, pl.ds(s * RPT, RPT)])


def _sc_aggregate(hp, src3, dst3):
    f = pl.kernel(
        _sc_aggregate_body,
        out_type=jax.ShapeDtypeStruct((NC, NPAD, D), jnp.float32),
        mesh=_mesh(),
        scratch_types=[
            pltpu.VMEM((NCHUNK, CH), jnp.int32),
            pltpu.VMEM((NCHUNK, CH), jnp.int32),
            pltpu.VMEM((CH, D), jnp.float32),
            pltpu.VMEM_SHARED((NPAD, D), jnp.float32),
        ] + [pltpu.SemaphoreType.DMA] * 2,
    )
    return f(hp, src3, dst3)


_BR = 1024
_GRID = NPAD // _BR


def _tc_pre_body(x_ref, w_ref, d0_ref, d1_ref, out_ref, dis_ref):
    deg = d0_ref[...] + d1_ref[...] + 1.0
    dis = lax.rsqrt(deg)
    h = jnp.dot(x_ref[...], w_ref[...], preferred_element_type=jnp.float32)
    out_ref[...] = h * dis
    dis_ref[...] = dis


def _layer_post(p0, p1, hp, dis, b, g, be):
    z = (p0 + p1 + hp) * dis + b
    t = jnp.tanh(z)
    mu = jnp.mean(t, axis=-1, keepdims=True)
    var = jnp.mean((t - mu) ** 2, axis=-1, keepdims=True)
    return (t - mu) / jnp.sqrt(var + EPS) * g + be


def _tc_mid_body(p0_ref, p1_ref, hp_ref, dis_ref,
                 b_ref, g_ref, be_ref, w_ref, out_ref):
    dis = dis_ref[...]
    y = _layer_post(p0_ref[...], p1_ref[...], hp_ref[...], dis,
                    b_ref[...], g_ref[...], be_ref[...])
    out_ref[...] = jnp.dot(y, w_ref[...],
                           preferred_element_type=jnp.float32) * dis


def _tc_final_body(p0_ref, p1_ref, hp_ref, dis_ref,
                   b_ref, g_ref, be_ref, out_ref):
    out_ref[...] = _layer_post(p0_ref[...], p1_ref[...], hp_ref[...],
                               dis_ref[...],
                               b_ref[...], g_ref[...], be_ref[...])


_ROWS = pl.BlockSpec((_BR, D), lambda i: (i, 0))
_FULLW = pl.BlockSpec((D, D), lambda i: (0, 0))
_VEC = pl.BlockSpec((1, D), lambda i: (0, 0))
_OUT = jax.ShapeDtypeStruct((NPAD, D), jnp.float32)


def _tc_pre(x, W, d0, d1):
    return pl.pallas_call(
        _tc_pre_body,
        grid=(_GRID,),
        in_specs=[_ROWS, _FULLW, _ROWS, _ROWS],
        out_specs=[_ROWS, _ROWS],
        out_shape=[_OUT, _OUT],
    )(x, W, d0, d1)


def _tc_mid(p0, p1, hp, dis, b, g, be, W):
    return pl.pallas_call(
        _tc_mid_body,
        grid=(_GRID,),
        in_specs=[_ROWS, _ROWS, _ROWS, _ROWS, _VEC, _VEC, _VEC, _FULLW],
        out_specs=_ROWS,
        out_shape=_OUT,
    )(p0, p1, hp, dis, b, g, be, W)


def _tc_final(p0, p1, hp, dis, b, g, be):
    return pl.pallas_call(
        _tc_final_body,
        grid=(_GRID,),
        in_specs=[_ROWS, _ROWS, _ROWS, _ROWS, _VEC, _VEC, _VEC],
        out_specs=_ROWS,
        out_shape=_OUT,
    )(p0, p1, hp, dis, b, g, be)


def kernel(x, edge_index, W1, b1, g1, be1, W2, b2, g2, be2):
    src3 = edge_index[0].reshape(NW, NCHUNK, CH)
    dst3 = edge_index[1].reshape(NW, NCHUNK, CH)
    xp = jnp.pad(x, ((0, NPAD - N), (0, 0)))
    b1, g1, be1 = b1.reshape(1, D), g1.reshape(1, D), be1.reshape(1, D)
    b2, g2, be2 = b2.reshape(1, D), g2.reshape(1, D), be2.reshape(1, D)

    ones = jnp.ones((NPAD, D), jnp.float32)
    degp = _sc_aggregate(ones, dst3, dst3)

    h1p, dis = _tc_pre(xp, W1, degp[0], degp[1])
    p = _sc_aggregate(h1p, src3, dst3)
    h2p = _tc_mid(p[0], p[1], h1p, dis, b1, g1, be1, W2)
    p2 = _sc_aggregate(h2p, src3, dst3)
    y = _tc_final(p2[0], p2[1], h2p, dis, b2, g2, be2)
    return y[:N]


# speedup vs baseline: 17.7812x; 1.2688x over previous
import jax
import jax.numpy as jnp
from jax import lax
from jax.experimental import pallas as pl
from jax.experimental.pallas import tpu as pltpu
from jax.experimental.pallas import tpu_sc as plsc

N = 10000
E = 320000
D = 128
EPS = 1e-5

NC = 2
NS = 16
NW = NC * NS
EPW = E // NW
CH = 80
NCHUNK = EPW // CH
NPAIR = (NCHUNK - 1) // 2
NPAD = 10240
RPT = NPAD // NS


def _mesh():
    return plsc.VectorSubcoreMesh(core_axis_name="c", subcore_axis_name="s",
                                  num_cores=NC, num_subcores=NS)


def _sc_aggregate_body(hp_hbm, src_hbm, dst_hbm, out_hbm,
                       idx_s, idx_d, rows_a, rows_b, agg,
                       gsa, gsb, ssa, ssb):
    c = lax.axis_index("c")
    s = lax.axis_index("s")
    wid = s * NC + c

    def zrow(i, carry):
        for q in range(D // 16):
            rows_a[i, pl.ds(q * 16, 16)] = jnp.zeros((16,), jnp.float32)
        return carry

    lax.fori_loop(0, CH, zrow, 0)
    for q in range(RPT // CH):
        pltpu.sync_copy(rows_a, agg.at[pl.ds(s * RPT + q * CH, CH)])
    off = pl.multiple_of(wid * EPW, 8)
    pltpu.sync_copy(src_hbm.at[pl.ds(off, EPW)], idx_s)
    pltpu.sync_copy(dst_hbm.at[wid], idx_d)
    plsc.subcore_barrier()

    def g_start(j, buf, sem):
        pltpu.async_copy(hp_hbm.at[idx_s.at[pl.ds(j * CH, CH)]], buf, sem)

    def g_wait(j, buf, sem):
        pltpu.make_async_copy(hp_hbm.at[idx_s.at[pl.ds(j * CH, CH)]],
                              buf, sem).wait()

    def s_start(j, buf, sem):
        pltpu.async_copy(buf, agg.at[idx_d.at[j]], sem, add=True)

    def s_wait(j, buf, sem):
        pltpu.make_async_copy(buf, agg.at[idx_d.at[j]], sem).wait()

    g_start(0, rows_a, gsa)

    def pair(t, carry):
        j0 = 2 * t
        g_wait(j0, rows_a, gsa)
        s_start(j0, rows_a, ssa)

        @pl.when(t > 0)
        def _():
            s_wait(j0 - 1, rows_b, ssb)

        g_start(j0 + 1, rows_b, gsb)
        g_wait(j0 + 1, rows_b, gsb)
        s_start(j0 + 1, rows_b, ssb)
        s_wait(j0, rows_a, ssa)
        g_start(j0 + 2, rows_a, gsa)
        return carry

    lax.fori_loop(0, NPAIR, pair, 0)
    last = NCHUNK - 1
    g_wait(last, rows_a, gsa)
    s_start(last, rows_a, ssa)
    s_wait(last - 1, rows_b, ssb)
    s_wait(last, rows_a, ssa)
    plsc.subcore_barrier()
    pltpu.sync_copy(agg.at[pl.ds(s * RPT, RPT)],
                    out_hbm.at[c, pl.ds(s * RPT, RPT)])


def _sc_aggregate(hp, src3, dst3):
    f = pl.kernel(
        _sc_aggregate_body,
        out_type=jax.ShapeDtypeStruct((NC, NPAD, D), jnp.float32),
        mesh=_mesh(),
        scratch_types=[
            pltpu.VMEM((EPW,), jnp.int32),
            pltpu.VMEM((NCHUNK, CH), jnp.int32),
            pltpu.VMEM((CH, D), jnp.float32),
            pltpu.VMEM((CH, D), jnp.float32),
            pltpu.VMEM_SHARED((NPAD, D), jnp.float32),
        ] + [pltpu.SemaphoreType.DMA] * 4,
    )
    return f(hp, src3, dst3)


_BR = 1024
_GRID = NPAD // _BR


def _tc_pre_body(x_ref, w_ref, d0_ref, d1_ref, out_ref, dis_ref):
    deg = d0_ref[...] + d1_ref[...] + 1.0
    dis = lax.rsqrt(deg)
    h = jnp.dot(x_ref[...], w_ref[...], preferred_element_type=jnp.float32)
    out_ref[...] = h * dis
    dis_ref[...] = dis


def _layer_post(p0, p1, hp, dis, b, g, be):
    z = (p0 + p1 + hp) * dis + b
    t = jnp.tanh(z)
    mu = jnp.mean(t, axis=-1, keepdims=True)
    var = jnp.mean((t - mu) ** 2, axis=-1, keepdims=True)
    return (t - mu) / jnp.sqrt(var + EPS) * g + be


def _tc_mid_body(p0_ref, p1_ref, hp_ref, dis_ref,
                 b_ref, g_ref, be_ref, w_ref, out_ref):
    dis = dis_ref[...]
    y = _layer_post(p0_ref[...], p1_ref[...], hp_ref[...], dis,
                    b_ref[...], g_ref[...], be_ref[...])
    out_ref[...] = jnp.dot(y, w_ref[...],
                           preferred_element_type=jnp.float32) * dis


def _tc_final_body(p0_ref, p1_ref, hp_ref, dis_ref,
                   b_ref, g_ref, be_ref, out_ref):
    out_ref[...] = _layer_post(p0_ref[...], p1_ref[...], hp_ref[...],
                               dis_ref[...],
                               b_ref[...], g_ref[...], be_ref[...])


_ROWS = pl.BlockSpec((_BR, D), lambda i: (i, 0))
_FULLW = pl.BlockSpec((D, D), lambda i: (0, 0))
_VEC = pl.BlockSpec((1, D), lambda i: (0, 0))
_OUT = jax.ShapeDtypeStruct((NPAD, D), jnp.float32)


def _tc_pre(x, W, d0, d1):
    return pl.pallas_call(
        _tc_pre_body,
        grid=(_GRID,),
        in_specs=[_ROWS, _FULLW, _ROWS, _ROWS],
        out_specs=[_ROWS, _ROWS],
        out_shape=[_OUT, _OUT],
    )(x, W, d0, d1)


def _tc_mid(p0, p1, hp, dis, b, g, be, W):
    return pl.pallas_call(
        _tc_mid_body,
        grid=(_GRID,),
        in_specs=[_ROWS, _ROWS, _ROWS, _ROWS, _VEC, _VEC, _VEC, _FULLW],
        out_specs=_ROWS,
        out_shape=_OUT,
    )(p0, p1, hp, dis, b, g, be, W)


def _tc_final(p0, p1, hp, dis, b, g, be):
    return pl.pallas_call(
        _tc_final_body,
        grid=(_GRID,),
        in_specs=[_ROWS, _ROWS, _ROWS, _ROWS, _VEC, _VEC, _VEC],
        out_specs=_ROWS,
        out_shape=_OUT,
    )(p0, p1, hp, dis, b, g, be)


def kernel(x, edge_index, W1, b1, g1, be1, W2, b2, g2, be2):
    src3 = edge_index[0]
    dst3 = edge_index[1].reshape(NW, NCHUNK, CH)
    xp = jnp.pad(x, ((0, NPAD - N), (0, 0)))
    b1, g1, be1 = b1.reshape(1, D), g1.reshape(1, D), be1.reshape(1, D)
    b2, g2, be2 = b2.reshape(1, D), g2.reshape(1, D), be2.reshape(1, D)

    ones = jnp.ones((NPAD, D), jnp.float32)
    degp = _sc_aggregate(ones, edge_index[1], dst3)

    h1p, dis = _tc_pre(xp, W1, degp[0], degp[1])
    p = _sc_aggregate(h1p, src3, dst3)
    h2p = _tc_mid(p[0], p[1], h1p, dis, b1, g1, be1, W2)
    p2 = _sc_aggregate(h2p, src3, dst3)
    y = _tc_final(p2[0], p2[1], h2p, dis, b2, g2, be2)
    return y[:N]


# speedup vs baseline: 21.1255x; 1.1881x over previous
import jax
import jax.numpy as jnp
from jax import lax
from jax.experimental import pallas as pl
from jax.experimental.pallas import tpu as pltpu
from jax.experimental.pallas import tpu_sc as plsc

N = 10000
E = 320000
D = 128
EPS = 1e-5

NC = 2
NS = 16
NW = NC * NS
EPW = E // NW
CH = 80
NCHUNK = EPW // CH
NPAIR = (NCHUNK - 1) // 2
NPAD = 10240
RPT = NPAD // NS


def _mesh():
    return plsc.VectorSubcoreMesh(core_axis_name="c", subcore_axis_name="s",
                                  num_cores=NC, num_subcores=NS)


def _sc_degree_body(dst_hbm, out_hbm, idx_d, obuf, hist, *sems):
    c = lax.axis_index("c")
    s = lax.axis_index("s")
    wid = s * NC + c

    def fill(val):
        def go(i, carry):
            for q in range(D // 16):
                obuf[i, pl.ds(q * 16, 16)] = jnp.full((16,), val, jnp.float32)
            return carry
        lax.fori_loop(0, CH, go, 0)

    fill(0.0)
    for q in range(RPT // CH):
        pltpu.sync_copy(obuf, hist.at[pl.ds(s * RPT + q * CH, CH)])
    fill(1.0)
    pltpu.sync_copy(dst_hbm.at[wid], idx_d)
    plsc.subcore_barrier()

    def group(t, carry):
        for b in range(len(sems)):
            j = t * len(sems) + b
            pltpu.async_copy(obuf, hist.at[idx_d.at[j]], sems[b], add=True)
        for b in range(len(sems)):
            j = t * len(sems) + b
            pltpu.make_async_copy(obuf, hist.at[idx_d.at[j]],
                                  sems[b]).wait()
        return carry

    lax.fori_loop(0, NCHUNK // 5, group, 0)
    plsc.subcore_barrier()
    pltpu.sync_copy(hist.at[pl.ds(s * RPT, RPT)],
                    out_hbm.at[c, pl.ds(s * RPT, RPT)])


def _sc_degree(dst3):
    f = pl.kernel(
        _sc_degree_body,
        out_type=jax.ShapeDtypeStruct((NC, NPAD, D), jnp.float32),
        mesh=_mesh(),
        scratch_types=[
            pltpu.VMEM((NCHUNK, CH), jnp.int32),
            pltpu.VMEM((CH, D), jnp.float32),
            pltpu.VMEM_SHARED((NPAD, D), jnp.float32),
        ] + [pltpu.SemaphoreType.DMA] * 5,
    )
    return f(dst3)


def _sc_aggregate_body(hp_hbm, src_hbm, dst_hbm, out_hbm,
                       idx_s, idx_d, rows_a, rows_b, agg,
                       gsa, gsb, ssa, ssb):
    c = lax.axis_index("c")
    s = lax.axis_index("s")
    wid = s * NC + c

    def zrow(i, carry):
        for q in range(D // 16):
            rows_a[i, pl.ds(q * 16, 16)] = jnp.zeros((16,), jnp.float32)
        return carry

    lax.fori_loop(0, CH, zrow, 0)
    for q in range(RPT // CH):
        pltpu.sync_copy(rows_a, agg.at[pl.ds(s * RPT + q * CH, CH)])
    off = pl.multiple_of(wid * EPW, 8)
    pltpu.sync_copy(src_hbm.at[pl.ds(off, EPW)], idx_s)
    pltpu.sync_copy(dst_hbm.at[wid], idx_d)
    plsc.subcore_barrier()

    def g_start(j, buf, sem):
        pltpu.async_copy(hp_hbm.at[idx_s.at[pl.ds(j * CH, CH)]], buf, sem)

    def g_wait(j, buf, sem):
        pltpu.make_async_copy(hp_hbm.at[idx_s.at[pl.ds(j * CH, CH)]],
                              buf, sem).wait()

    def s_start(j, buf, sem):
        pltpu.async_copy(buf, agg.at[idx_d.at[j]], sem, add=True)

    def s_wait(j, buf, sem):
        pltpu.make_async_copy(buf, agg.at[idx_d.at[j]], sem).wait()

    g_start(0, rows_a, gsa)

    def pair(t, carry):
        j0 = 2 * t
        g_wait(j0, rows_a, gsa)
        s_start(j0, rows_a, ssa)

        @pl.when(t > 0)
        def _():
            s_wait(j0 - 1, rows_b, ssb)

        g_start(j0 + 1, rows_b, gsb)
        g_wait(j0 + 1, rows_b, gsb)
        s_start(j0 + 1, rows_b, ssb)
        s_wait(j0, rows_a, ssa)
        g_start(j0 + 2, rows_a, gsa)
        return carry

    lax.fori_loop(0, NPAIR, pair, 0)
    last = NCHUNK - 1
    g_wait(last, rows_a, gsa)
    s_start(last, rows_a, ssa)
    s_wait(last - 1, rows_b, ssb)
    s_wait(last, rows_a, ssa)
    plsc.subcore_barrier()
    pltpu.sync_copy(agg.at[pl.ds(s * RPT, RPT)],
                    out_hbm.at[c, pl.ds(s * RPT, RPT)])


def _sc_aggregate(hp, src3, dst3):
    f = pl.kernel(
        _sc_aggregate_body,
        out_type=jax.ShapeDtypeStruct((NC, NPAD, D), jnp.float32),
        mesh=_mesh(),
        scratch_types=[
            pltpu.VMEM((EPW,), jnp.int32),
            pltpu.VMEM((NCHUNK, CH), jnp.int32),
            pltpu.VMEM((CH, D), jnp.float32),
            pltpu.VMEM((CH, D), jnp.float32),
            pltpu.VMEM_SHARED((NPAD, D), jnp.float32),
        ] + [pltpu.SemaphoreType.DMA] * 4,
    )
    return f(hp, src3, dst3)


_BR = 1024
_GRID = NPAD // _BR


def _tc_pre_body(x_ref, w_ref, d0_ref, d1_ref, out_ref, dis_ref):
    deg = d0_ref[...] + d1_ref[...] + 1.0
    dis = lax.rsqrt(deg)
    h = jnp.dot(x_ref[...], w_ref[...], preferred_element_type=jnp.float32)
    out_ref[...] = h * dis
    dis_ref[...] = dis


def _layer_post(p0, p1, hp, dis, b, g, be):
    z = (p0 + p1 + hp) * dis + b
    t = jnp.tanh(z)
    mu = jnp.mean(t, axis=-1, keepdims=True)
    var = jnp.mean((t - mu) ** 2, axis=-1, keepdims=True)
    return (t - mu) / jnp.sqrt(var + EPS) * g + be


def _tc_mid_body(p0_ref, p1_ref, hp_ref, dis_ref,
                 b_ref, g_ref, be_ref, w_ref, out_ref):
    dis = dis_ref[...]
    y = _layer_post(p0_ref[...], p1_ref[...], hp_ref[...], dis,
                    b_ref[...], g_ref[...], be_ref[...])
    out_ref[...] = jnp.dot(y, w_ref[...],
                           preferred_element_type=jnp.float32) * dis


def _tc_final_body(p0_ref, p1_ref, hp_ref, dis_ref,
                   b_ref, g_ref, be_ref, out_ref):
    out_ref[...] = _layer_post(p0_ref[...], p1_ref[...], hp_ref[...],
                               dis_ref[...],
                               b_ref[...], g_ref[...], be_ref[...])


_ROWS = pl.BlockSpec((_BR, D), lambda i: (i, 0))
_FULLW = pl.BlockSpec((D, D), lambda i: (0, 0))
_VEC = pl.BlockSpec((1, D), lambda i: (0, 0))
_OUT = jax.ShapeDtypeStruct((NPAD, D), jnp.float32)


def _tc_pre(x, W, d0, d1):
    return pl.pallas_call(
        _tc_pre_body,
        grid=(_GRID,),
        in_specs=[_ROWS, _FULLW, _ROWS, _ROWS],
        out_specs=[_ROWS, _ROWS],
        out_shape=[_OUT, _OUT],
    )(x, W, d0, d1)


def _tc_mid(p0, p1, hp, dis, b, g, be, W):
    return pl.pallas_call(
        _tc_mid_body,
        grid=(_GRID,),
        in_specs=[_ROWS, _ROWS, _ROWS, _ROWS, _VEC, _VEC, _VEC, _FULLW],
        out_specs=_ROWS,
        out_shape=_OUT,
    )(p0, p1, hp, dis, b, g, be, W)


def _tc_final(p0, p1, hp, dis, b, g, be):
    return pl.pallas_call(
        _tc_final_body,
        grid=(_GRID,),
        in_specs=[_ROWS, _ROWS, _ROWS, _ROWS, _VEC, _VEC, _VEC],
        out_specs=_ROWS,
        out_shape=_OUT,
    )(p0, p1, hp, dis, b, g, be)


def kernel(x, edge_index, W1, b1, g1, be1, W2, b2, g2, be2):
    src3 = edge_index[0]
    dst3 = edge_index[1].reshape(NW, NCHUNK, CH)
    xp = jnp.pad(x, ((0, NPAD - N), (0, 0)))
    b1, g1, be1 = b1.reshape(1, D), g1.reshape(1, D), be1.reshape(1, D)
    b2, g2, be2 = b2.reshape(1, D), g2.reshape(1, D), be2.reshape(1, D)

    degp = _sc_degree(dst3)

    h1p, dis = _tc_pre(xp, W1, degp[0], degp[1])
    p = _sc_aggregate(h1p, src3, dst3)
    h2p = _tc_mid(p[0], p[1], h1p, dis, b1, g1, be1, W2)
    p2 = _sc_aggregate(h2p, src3, dst3)
    y = _tc_final(p2[0], p2[1], h2p, dis, b2, g2, be2)
    return y[:N]


# speedup vs baseline: 22.2473x; 1.0531x over previous
import jax
import jax.numpy as jnp
from jax import lax
from jax.experimental import pallas as pl
from jax.experimental.pallas import tpu as pltpu
from jax.experimental.pallas import tpu_sc as plsc

N = 10000
E = 320000
D = 128
EPS = 1e-5

NC = 2
NS = 16
NW = NC * NS
EPW = E // NW
CH = 96
EPWP = 10080
NCHUNK = EPWP // CH
NPAIR = (NCHUNK - 1) // 2
NPAD = 10240
RPT = NPAD // NS


def _mesh():
    return plsc.VectorSubcoreMesh(core_axis_name="c", subcore_axis_name="s",
                                  num_cores=NC, num_subcores=NS)


def _sc_degree_body(dst_hbm, out_hbm, idx_d, obuf, hist, *sems):
    c = lax.axis_index("c")
    s = lax.axis_index("s")
    wid = s * NC + c

    def fill(val):
        def go(i, carry):
            for q in range(D // 16):
                obuf[i, pl.ds(q * 16, 16)] = jnp.full((16,), val, jnp.float32)
            return carry
        lax.fori_loop(0, CH, go, 0)

    fill(0.0)
    for q in range(RPT // CH):
        pltpu.sync_copy(obuf, hist.at[pl.ds(s * RPT + q * CH, CH)])
    fill(1.0)
    pltpu.sync_copy(dst_hbm.at[wid], idx_d)
    plsc.subcore_barrier()

    def group(t, carry):
        for b in range(len(sems)):
            j = t * len(sems) + b
            pltpu.async_copy(obuf, hist.at[idx_d.at[j]], sems[b], add=True)
        for b in range(len(sems)):
            j = t * len(sems) + b
            pltpu.make_async_copy(obuf, hist.at[idx_d.at[j]],
                                  sems[b]).wait()
        return carry

    lax.fori_loop(0, NCHUNK // 5, group, 0)
    plsc.subcore_barrier()
    pltpu.sync_copy(hist.at[pl.ds(s * RPT, RPT)],
                    out_hbm.at[c, pl.ds(s * RPT, RPT)])


def _sc_degree(dst3):
    f = pl.kernel(
        _sc_degree_body,
        out_type=jax.ShapeDtypeStruct((NC, NPAD, D), jnp.float32),
        mesh=_mesh(),
        scratch_types=[
            pltpu.VMEM((NCHUNK, CH), jnp.int32),
            pltpu.VMEM((CH, D), jnp.float32),
            pltpu.VMEM_SHARED((NPAD, D), jnp.float32),
        ] + [pltpu.SemaphoreType.DMA] * 5,
    )
    return f(dst3)


def _sc_aggregate_body(hp_hbm, src_hbm, dst_hbm, out_hbm,
                       idx_s, idx_d, rows_a, rows_b, agg,
                       gsa, gsb, ssa, ssb):
    c = lax.axis_index("c")
    s = lax.axis_index("s")
    wid = s * NC + c

    def zrow(i, carry):
        for q in range(D // 16):
            rows_a[i, pl.ds(q * 16, 16)] = jnp.zeros((16,), jnp.float32)
        return carry

    lax.fori_loop(0, CH, zrow, 0)
    for q in range(RPT // CH):
        pltpu.sync_copy(rows_a, agg.at[pl.ds(s * RPT + q * CH, CH)])
    off = pl.multiple_of(wid * EPWP, 8)
    pltpu.sync_copy(src_hbm.at[pl.ds(off, EPWP)], idx_s)
    pltpu.sync_copy(dst_hbm.at[wid], idx_d)
    plsc.subcore_barrier()

    def g_start(j, buf, sem):
        pltpu.async_copy(hp_hbm.at[idx_s.at[pl.ds(j * CH, CH)]], buf, sem)

    def g_wait(j, buf, sem):
        pltpu.make_async_copy(hp_hbm.at[idx_s.at[pl.ds(j * CH, CH)]],
                              buf, sem).wait()

    def s_start(j, buf, sem):
        pltpu.async_copy(buf, agg.at[idx_d.at[j]], sem, add=True)

    def s_wait(j, buf, sem):
        pltpu.make_async_copy(buf, agg.at[idx_d.at[j]], sem).wait()

    g_start(0, rows_a, gsa)

    def pair(t, carry):
        j0 = 2 * t
        g_wait(j0, rows_a, gsa)
        s_start(j0, rows_a, ssa)

        @pl.when(t > 0)
        def _():
            s_wait(j0 - 1, rows_b, ssb)

        g_start(j0 + 1, rows_b, gsb)
        g_wait(j0 + 1, rows_b, gsb)
        s_start(j0 + 1, rows_b, ssb)
        s_wait(j0, rows_a, ssa)
        g_start(j0 + 2, rows_a, gsa)
        return carry

    lax.fori_loop(0, NPAIR, pair, 0)
    last = NCHUNK - 1
    g_wait(last, rows_a, gsa)
    s_start(last, rows_a, ssa)
    s_wait(last - 1, rows_b, ssb)
    s_wait(last, rows_a, ssa)
    plsc.subcore_barrier()
    pltpu.sync_copy(agg.at[pl.ds(s * RPT, RPT)],
                    out_hbm.at[c, pl.ds(s * RPT, RPT)])


def _sc_aggregate(hp, src3, dst3):
    f = pl.kernel(
        _sc_aggregate_body,
        out_type=jax.ShapeDtypeStruct((NC, NPAD, D), jnp.float32),
        mesh=_mesh(),
        scratch_types=[
            pltpu.VMEM((EPWP,), jnp.int32),
            pltpu.VMEM((NCHUNK, CH), jnp.int32),
            pltpu.VMEM((CH, D), jnp.float32),
            pltpu.VMEM((CH, D), jnp.float32),
            pltpu.VMEM_SHARED((NPAD, D), jnp.float32),
        ] + [pltpu.SemaphoreType.DMA] * 4,
    )
    return f(hp, src3, dst3)


_BR = 1024
_GRID = NPAD // _BR


def _tc_pre_body(x_ref, w_ref, d0_ref, d1_ref, out_ref, dis_ref):
    deg = d0_ref[...] + d1_ref[...] + 1.0
    dis = lax.rsqrt(deg)
    h = jnp.dot(x_ref[...], w_ref[...], preferred_element_type=jnp.float32)
    out_ref[...] = h * dis
    dis_ref[...] = dis


def _layer_post(p0, p1, hp, dis, b, g, be):
    z = (p0 + p1 + hp) * dis + b
    t = jnp.tanh(z)
    mu = jnp.mean(t, axis=-1, keepdims=True)
    var = jnp.mean((t - mu) ** 2, axis=-1, keepdims=True)
    return (t - mu) / jnp.sqrt(var + EPS) * g + be


def _tc_mid_body(p0_ref, p1_ref, hp_ref, dis_ref,
                 b_ref, g_ref, be_ref, w_ref, out_ref):
    dis = dis_ref[...]
    y = _layer_post(p0_ref[...], p1_ref[...], hp_ref[...], dis,
                    b_ref[...], g_ref[...], be_ref[...])
    out_ref[...] = jnp.dot(y, w_ref[...],
                           preferred_element_type=jnp.float32) * dis


def _tc_final_body(p0_ref, p1_ref, hp_ref, dis_ref,
                   b_ref, g_ref, be_ref, out_ref):
    out_ref[...] = _layer_post(p0_ref[...], p1_ref[...], hp_ref[...],
                               dis_ref[...],
                               b_ref[...], g_ref[...], be_ref[...])


_ROWS = pl.BlockSpec((_BR, D), lambda i: (i, 0))
_FULLW = pl.BlockSpec((D, D), lambda i: (0, 0))
_VEC = pl.BlockSpec((1, D), lambda i: (0, 0))
_OUT = jax.ShapeDtypeStruct((NPAD, D), jnp.float32)


def _tc_pre(x, W, d0, d1):
    return pl.pallas_call(
        _tc_pre_body,
        grid=(_GRID,),
        in_specs=[_ROWS, _FULLW, _ROWS, _ROWS],
        out_specs=[_ROWS, _ROWS],
        out_shape=[_OUT, _OUT],
    )(x, W, d0, d1)


def _tc_mid(p0, p1, hp, dis, b, g, be, W):
    return pl.pallas_call(
        _tc_mid_body,
        grid=(_GRID,),
        in_specs=[_ROWS, _ROWS, _ROWS, _ROWS, _VEC, _VEC, _VEC, _FULLW],
        out_specs=_ROWS,
        out_shape=_OUT,
    )(p0, p1, hp, dis, b, g, be, W)


def _tc_final(p0, p1, hp, dis, b, g, be):
    return pl.pallas_call(
        _tc_final_body,
        grid=(_GRID,),
        in_specs=[_ROWS, _ROWS, _ROWS, _ROWS, _VEC, _VEC, _VEC],
        out_specs=_ROWS,
        out_shape=_OUT,
    )(p0, p1, hp, dis, b, g, be)


def kernel(x, edge_index, W1, b1, g1, be1, W2, b2, g2, be2):
    pad_idx = N + (jnp.arange(NW, dtype=jnp.int32)[:, None] * 8
                   + jnp.arange(EPWP - EPW, dtype=jnp.int32)[None, :]) % (NPAD - N)
    def padw(a):
        return jnp.concatenate([a.reshape(NW, EPW), pad_idx], axis=1)
    src3 = padw(edge_index[0]).reshape(-1)
    dst3 = padw(edge_index[1]).reshape(NW, NCHUNK, CH)
    xp = jnp.pad(x, ((0, NPAD - N), (0, 0)))
    b1, g1, be1 = b1.reshape(1, D), g1.reshape(1, D), be1.reshape(1, D)
    b2, g2, be2 = b2.reshape(1, D), g2.reshape(1, D), be2.reshape(1, D)

    degp = _sc_degree(dst3)

    h1p, dis = _tc_pre(xp, W1, degp[0], degp[1])
    p = _sc_aggregate(h1p, src3, dst3)
    h2p = _tc_mid(p[0], p[1], h1p, dis, b1, g1, be1, W2)
    p2 = _sc_aggregate(h2p, src3, dst3)
    y = _tc_final(p2[0], p2[1], h2p, dis, b2, g2, be2)
    return y[:N]
